# baseline (speedup 1.0000x reference)
import functools

import jax
import jax.numpy as jnp
from jax import lax
from jax.experimental import pallas as pl
from jax.experimental.pallas import tpu as pltpu
from jax.experimental.pallas import tpu_sc as plsc

_N = 10000
_E = 320000
_CH = 32
_NPAD = 10240
_CHUNK = 128
_NC, _NS = 2, 16
_NW = _NC * _NS
_EPAD = _NW * 10240
_CPT = _EPAD // (_NW * _CHUNK)
_NBUF = 4
_NGRP = _CPT // _NBUF
_ROWS_PER_TILE = _NPAD // _NS


def _sc_aggregate(with_deg):
    out_type = [jax.ShapeDtypeStruct((_NC, _NPAD, _CH), jnp.float32)]
    scratch = [
        pltpu.VMEM((_CPT, _CHUNK), jnp.int32),
        pltpu.VMEM((_CPT, _CHUNK), jnp.int32),
    ]
    scratch += [pltpu.VMEM((_CHUNK, _CH), jnp.float32)] * _NBUF
    scratch += [pltpu.VMEM_SHARED((_NPAD, _CH), jnp.float32)]
    scratch += [pltpu.SemaphoreType.DMA] * (2 * _NBUF)
    if with_deg:
        out_type.append(jax.ShapeDtypeStruct((_NC, _NPAD, _CH), jnp.float32))
        scratch += [
            pltpu.VMEM((_CHUNK, _CH), jnp.float32),
            pltpu.VMEM_SHARED((_NPAD, _CH), jnp.float32),
            pltpu.SemaphoreType.DMA,
        ]

    mesh = plsc.VectorSubcoreMesh(core_axis_name="c", subcore_axis_name="s")

    def body(xs, src3d, dst3d, zeros, *rest):
        if with_deg:
            (ones_hbm, agg_out, deg_out, src_v, dst_v,
             r0b, r1b, r2b, r3b, acc,
             g0, g1, g2, g3, s0, s1, s2, s3,
             ones_v, dacc, sem_d) = rest
        else:
            (agg_out, src_v, dst_v,
             r0b, r1b, r2b, r3b, acc,
             g0, g1, g2, g3, s0, s1, s2, s3) = rest
        rows = (r0b, r1b, r2b, r3b)
        sem_g = (g0, g1, g2, g3)
        sem_s = (s0, s1, s2, s3)

        c = lax.axis_index("c")
        s = lax.axis_index("s")
        w = s * _NC + c

        r0 = s * _ROWS_PER_TILE
        pltpu.sync_copy(zeros.at[pl.ds(r0, _ROWS_PER_TILE)],
                        acc.at[pl.ds(r0, _ROWS_PER_TILE)])
        if with_deg:
            pltpu.sync_copy(zeros.at[pl.ds(r0, _ROWS_PER_TILE)],
                            dacc.at[pl.ds(r0, _ROWS_PER_TILE)])
            pltpu.sync_copy(ones_hbm, ones_v)

        pltpu.sync_copy(src3d.at[w], src_v)
        pltpu.sync_copy(dst3d.at[w], dst_v)

        plsc.subcore_barrier()

        def gth(j, b):
            pltpu.async_copy(xs.at[src_v.at[j]], rows[b], sem_g[b])

        def wait_g(j, b):
            pltpu.make_async_copy(xs.at[src_v.at[j]], rows[b], sem_g[b]).wait()

        def sct(j, b):
            pltpu.async_copy(rows[b], acc.at[dst_v.at[j]], sem_s[b], add=True)
            if with_deg:
                pltpu.async_copy(ones_v, dacc.at[dst_v.at[j]], sem_d, add=True)

        def wait_s(j, b):
            pltpu.make_async_copy(rows[b], acc.at[dst_v.at[j]], sem_s[b]).wait()

        for b in range(_NBUF - 1):
            gth(b, b)
        wait_g(0, 0); sct(0, 0); gth(3, 3)
        for j in range(1, _NBUF):
            wait_g(j, j); sct(j, j)
            wait_s(j - 1, j - 1); gth(j + 3, (j - 1) % _NBUF)

        def group(i, carry):
            j0 = i * _NBUF
            for b in range(_NBUF):
                j = j0 + b
                wait_g(j, b); sct(j, b)
                wait_s(j - 1, (b - 1) % _NBUF)
                gth(j + _NBUF - 1, (b - 1) % _NBUF)
            return carry

        lax.fori_loop(1, _NGRP - 1, group, 0)

        jl = (_NGRP - 1) * _NBUF
        wait_g(jl, 0); sct(jl, 0)
        wait_s(jl - 1, 3); gth(jl + 3, 3)
        for b in range(1, _NBUF):
            wait_g(jl + b, b); sct(jl + b, b)
            wait_s(jl + b - 1, b - 1)
        wait_s(_CPT - 1, 3)

        if with_deg:
            def drain(j, carry):
                pltpu.make_async_copy(ones_v, dacc.at[dst_v.at[j]],
                                      sem_d).wait()
                return carry

            lax.fori_loop(0, _CPT, drain, 0)

        plsc.subcore_barrier()

        pltpu.sync_copy(acc.at[pl.ds(r0, _ROWS_PER_TILE)],
                        agg_out.at[c, pl.ds(r0, _ROWS_PER_TILE)])
        if with_deg:
            pltpu.sync_copy(dacc.at[pl.ds(r0, _ROWS_PER_TILE)],
                            deg_out.at[c, pl.ds(r0, _ROWS_PER_TILE)])

    return pl.kernel(body, out_type=out_type, mesh=mesh, scratch_types=scratch,
                     compiler_params=pltpu.CompilerParams(
                         use_tc_tiling_on_sc=False))


def _tc_proj(x_ref, wl_ref, wr_ref, xs_ref, hr_ref):
    x = x_ref[...]
    xs_ref[...] = jnp.dot(x, wl_ref[...], preferred_element_type=jnp.float32)
    hr_ref[...] = jnp.dot(x, wr_ref[...], preferred_element_type=jnp.float32)


def _load_partials(agg_hbm, deg_hbm, agg_v, deg_v, sem_a, sem_d):
    pltpu.async_copy(agg_hbm, agg_v, sem_a)
    pltpu.async_copy(deg_hbm, deg_v, sem_d)
    pltpu.make_async_copy(agg_hbm, agg_v, sem_a).wait()
    pltpu.make_async_copy(deg_hbm, deg_v, sem_d).wait()
    agg = agg_v[0, :_N, :] + agg_v[1, :_N, :]
    deg = deg_v[0, :_N, :] + deg_v[1, :_N, :]
    return agg, deg


def _tc_combine_proj(agg_hbm, deg_hbm, hr_ref, b_ref, wl_ref, wr_ref,
                     xs_ref, hrn_ref, agg_v, deg_v, sem_a, sem_d):
    agg, deg = _load_partials(agg_hbm, deg_hbm, agg_v, deg_v, sem_a, sem_d)
    inv = 1.0 / jnp.maximum(deg, 1.0)
    h = jnp.maximum(agg * inv + b_ref[...] + hr_ref[...], 0.0)
    xs_ref[...] = jnp.dot(h, wl_ref[...], preferred_element_type=jnp.float32)
    hrn_ref[...] = jnp.dot(h, wr_ref[...], preferred_element_type=jnp.float32)


def _tc_combine_final(agg_hbm, deg_hbm, hr_ref, b_ref, wo_ref, bo_ref,
                      out_ref, agg_v, deg_v, sem_a, sem_d):
    agg, deg = _load_partials(agg_hbm, deg_hbm, agg_v, deg_v, sem_a, sem_d)
    inv = 1.0 / jnp.maximum(deg, 1.0)
    h = jnp.maximum(agg * inv + b_ref[...] + hr_ref[...], 0.0)
    out_ref[...] = (jnp.dot(h, wo_ref[...], preferred_element_type=jnp.float32)
                    + bo_ref[...])


_f32 = jnp.float32


@jax.jit
def kernel(x, edge_index, W1l, b1, W1r, W2l, b2, W2r, W3l, b3, W3r, Wo, bo):
    pad = _EPAD - _E
    pad_iota = jnp.arange(pad, dtype=jnp.int32)
    src2d = jnp.concatenate(
        [edge_index[0], pad_iota % _N]).reshape(_NW, _CPT, _CHUNK)
    dst2d = jnp.concatenate(
        [edge_index[1], _N + pad_iota % (_NPAD - _N)]).reshape(
            _NW, _CPT, _CHUNK)
    zeros = jnp.zeros((_NPAD, _CH), _f32)
    ones = jnp.ones((_CHUNK, _CH), _f32)

    two_proj = pl.pallas_call(
        _tc_proj,
        out_shape=[jax.ShapeDtypeStruct((_N, _CH), _f32),
                   jax.ShapeDtypeStruct((_N, _CH), _f32)],
    )
    partial_specs = [pl.BlockSpec(memory_space=pl.ANY),
                     pl.BlockSpec(memory_space=pl.ANY)]
    vmem_spec = pl.BlockSpec(memory_space=pltpu.VMEM)
    partial_scratch = [
        pltpu.VMEM((_NC, _NPAD, _CH), _f32),
        pltpu.VMEM((_NC, _NPAD, _CH), _f32),
        pltpu.SemaphoreType.DMA,
        pltpu.SemaphoreType.DMA,
    ]
    combine_proj = pl.pallas_call(
        _tc_combine_proj,
        in_specs=partial_specs + [vmem_spec] * 4,
        out_shape=[jax.ShapeDtypeStruct((_N, _CH), _f32),
                   jax.ShapeDtypeStruct((_N, _CH), _f32)],
        scratch_shapes=partial_scratch,
    )
    combine_final = pl.pallas_call(
        _tc_combine_final,
        in_specs=partial_specs + [vmem_spec] * 4,
        out_shape=jax.ShapeDtypeStruct((_N, 128), _f32),
        scratch_shapes=partial_scratch,
    )

    agg_deg = _sc_aggregate(True)
    agg_only = _sc_aggregate(False)

    xs1, hr1 = two_proj(x, W1l.T, W1r.T)
    agg1, deg = agg_deg(xs1, src2d, dst2d, zeros, ones)
    xs2, hr2 = combine_proj(agg1, deg, hr1, b1.reshape(1, _CH), W2l.T, W2r.T)
    (agg2,) = agg_only(xs2, src2d, dst2d, zeros)
    xs3, hr3 = combine_proj(agg2, deg, hr2, b2.reshape(1, _CH), W3l.T, W3r.T)
    (agg3,) = agg_only(xs3, src2d, dst2d, zeros)
    out = combine_final(agg3, deg, hr3, b3.reshape(1, _CH), Wo.T,
                        bo.reshape(1, 128))
    return out


# speedup vs baseline: 1.0754x; 1.0754x over previous
import functools

import jax
import jax.numpy as jnp
from jax import lax
from jax.experimental import pallas as pl
from jax.experimental.pallas import tpu as pltpu
from jax.experimental.pallas import tpu_sc as plsc

_N = 10000
_E = 320000
_CH = 32
_NPAD = 10240
_CHUNK = 128
_NC, _NS = 2, 16
_NW = _NC * _NS
_EPAD = _NW * 10240
_CPT = _EPAD // (_NW * _CHUNK)
_NBUF = 8
_NGRP = _CPT // _NBUF
_ROWS_PER_TILE = _NPAD // _NS


def _sc_aggregate(with_deg):
    out_type = [jax.ShapeDtypeStruct((_NC, _NPAD, _CH), jnp.float32)]
    scratch = [
        pltpu.VMEM((_CPT, _CHUNK), jnp.int32),
        pltpu.VMEM((_CPT, _CHUNK), jnp.int32),
    ]
    scratch += [pltpu.VMEM((_CHUNK, _CH), jnp.float32)] * _NBUF
    scratch += [pltpu.VMEM_SHARED((_NPAD, _CH), jnp.float32)]
    scratch += [pltpu.SemaphoreType.DMA] * (2 * _NBUF)
    if with_deg:
        out_type.append(jax.ShapeDtypeStruct((_NC, _NPAD, _CH), jnp.float32))
        scratch += [
            pltpu.VMEM((_CHUNK, _CH), jnp.float32),
            pltpu.VMEM_SHARED((_NPAD, _CH), jnp.float32),
            pltpu.SemaphoreType.DMA,
        ]

    mesh = plsc.VectorSubcoreMesh(core_axis_name="c", subcore_axis_name="s")

    def body(xs, src3d, dst3d, zeros, *rest):
        if with_deg:
            ones_hbm, agg_out, deg_out = rest[:3]
            rest = rest[3:]
        else:
            (agg_out,) = rest[:1]
            rest = rest[1:]
        src_v, dst_v = rest[0], rest[1]
        rows = rest[2:2 + _NBUF]
        acc = rest[2 + _NBUF]
        sem_g = rest[3 + _NBUF:3 + 2 * _NBUF]
        sem_s = rest[3 + 2 * _NBUF:3 + 3 * _NBUF]
        if with_deg:
            ones_v, dacc, sem_d = rest[3 + 3 * _NBUF:]

        c = lax.axis_index("c")
        s = lax.axis_index("s")
        w = s * _NC + c

        r0 = s * _ROWS_PER_TILE
        pltpu.async_copy(zeros.at[pl.ds(r0, _ROWS_PER_TILE)],
                         acc.at[pl.ds(r0, _ROWS_PER_TILE)], sem_g[0])
        pltpu.async_copy(src3d.at[w], src_v, sem_g[1])
        pltpu.async_copy(dst3d.at[w], dst_v, sem_g[2])
        if with_deg:
            pltpu.async_copy(zeros.at[pl.ds(r0, _ROWS_PER_TILE)],
                             dacc.at[pl.ds(r0, _ROWS_PER_TILE)], sem_s[0])
            pltpu.async_copy(ones_hbm, ones_v, sem_s[1])
            pltpu.make_async_copy(zeros.at[pl.ds(r0, _ROWS_PER_TILE)],
                                  dacc.at[pl.ds(r0, _ROWS_PER_TILE)],
                                  sem_s[0]).wait()
            pltpu.make_async_copy(ones_hbm, ones_v, sem_s[1]).wait()
        pltpu.make_async_copy(zeros.at[pl.ds(r0, _ROWS_PER_TILE)],
                              acc.at[pl.ds(r0, _ROWS_PER_TILE)],
                              sem_g[0]).wait()
        pltpu.make_async_copy(src3d.at[w], src_v, sem_g[1]).wait()
        pltpu.make_async_copy(dst3d.at[w], dst_v, sem_g[2]).wait()

        plsc.subcore_barrier()

        def gth(j, b):
            pltpu.async_copy(xs.at[src_v.at[j]], rows[b], sem_g[b])

        def wait_g(j, b):
            pltpu.make_async_copy(xs.at[src_v.at[j]], rows[b], sem_g[b]).wait()

        def sct(j, b):
            pltpu.async_copy(rows[b], acc.at[dst_v.at[j]], sem_s[b], add=True)
            if with_deg:
                pltpu.async_copy(ones_v, dacc.at[dst_v.at[j]], sem_d, add=True)

        def wait_s(j, b):
            pltpu.make_async_copy(rows[b], acc.at[dst_v.at[j]], sem_s[b]).wait()

        for b in range(_NBUF - 1):
            gth(b, b)
        wait_g(0, 0); sct(0, 0); gth(_NBUF - 1, _NBUF - 1)
        for j in range(1, _NBUF):
            wait_g(j, j); sct(j, j)
            wait_s(j - 1, j - 1); gth(j + _NBUF - 1, (j - 1) % _NBUF)

        def group(i, carry):
            j0 = i * _NBUF
            for b in range(_NBUF):
                j = j0 + b
                wait_g(j, b); sct(j, b)
                wait_s(j - 1, (b - 1) % _NBUF)
                gth(j + _NBUF - 1, (b - 1) % _NBUF)
            return carry

        lax.fori_loop(1, _NGRP - 1, group, 0)

        jl = (_NGRP - 1) * _NBUF
        wait_g(jl, 0); sct(jl, 0)
        wait_s(jl - 1, _NBUF - 1); gth(jl + _NBUF - 1, _NBUF - 1)
        for b in range(1, _NBUF):
            wait_g(jl + b, b); sct(jl + b, b)
            wait_s(jl + b - 1, b - 1)
        wait_s(_CPT - 1, _NBUF - 1)

        if with_deg:
            def drain(j, carry):
                pltpu.make_async_copy(ones_v, dacc.at[dst_v.at[j]],
                                      sem_d).wait()
                return carry

            lax.fori_loop(0, _CPT, drain, 0)

        plsc.subcore_barrier()

        pltpu.sync_copy(acc.at[pl.ds(r0, _ROWS_PER_TILE)],
                        agg_out.at[c, pl.ds(r0, _ROWS_PER_TILE)])
        if with_deg:
            pltpu.sync_copy(dacc.at[pl.ds(r0, _ROWS_PER_TILE)],
                            deg_out.at[c, pl.ds(r0, _ROWS_PER_TILE)])

    return pl.kernel(body, out_type=out_type, mesh=mesh, scratch_types=scratch,
                     compiler_params=pltpu.CompilerParams(
                         use_tc_tiling_on_sc=False))


def _tc_proj(x_ref, wl_ref, wr_ref, xs_ref, hr_ref):
    x = x_ref[...]
    xs_ref[...] = jnp.dot(x, wl_ref[...], preferred_element_type=jnp.float32)
    hr_ref[...] = jnp.dot(x, wr_ref[...], preferred_element_type=jnp.float32)


def _tc_combine_proj(agg_ref, deg_ref, hr_ref, b_ref, wl_ref, wr_ref,
                     xs_ref, hrn_ref):
    agg = agg_ref[0, :_N, :] + agg_ref[1, :_N, :]
    deg = deg_ref[0, :_N, :] + deg_ref[1, :_N, :]
    inv = 1.0 / jnp.maximum(deg, 1.0)
    h = jnp.maximum(agg * inv + b_ref[...] + hr_ref[...], 0.0)
    xs_ref[...] = jnp.dot(h, wl_ref[...], preferred_element_type=jnp.float32)
    hrn_ref[...] = jnp.dot(h, wr_ref[...], preferred_element_type=jnp.float32)


def _tc_combine_final(agg_ref, deg_ref, hr_ref, b_ref, wo_ref, bo_ref,
                      out_ref):
    agg = agg_ref[0, :_N, :] + agg_ref[1, :_N, :]
    deg = deg_ref[0, :_N, :] + deg_ref[1, :_N, :]
    inv = 1.0 / jnp.maximum(deg, 1.0)
    h = jnp.maximum(agg * inv + b_ref[...] + hr_ref[...], 0.0)
    out_ref[...] = (jnp.dot(h, wo_ref[...], preferred_element_type=jnp.float32)
                    + bo_ref[...])


_f32 = jnp.float32


@jax.jit
def kernel(x, edge_index, W1l, b1, W1r, W2l, b2, W2r, W3l, b3, W3r, Wo, bo):
    pad = _EPAD - _E
    pad_iota = jnp.arange(pad, dtype=jnp.int32)
    src2d = jnp.concatenate(
        [edge_index[0], pad_iota % _N]).reshape(_NW, _CPT, _CHUNK)
    dst2d = jnp.concatenate(
        [edge_index[1], _N + pad_iota % (_NPAD - _N)]).reshape(
            _NW, _CPT, _CHUNK)
    zeros = jnp.zeros((_NPAD, _CH), _f32)
    ones = jnp.ones((_CHUNK, _CH), _f32)

    two_proj = pl.pallas_call(
        _tc_proj,
        out_shape=[jax.ShapeDtypeStruct((_N, _CH), _f32),
                   jax.ShapeDtypeStruct((_N, _CH), _f32)],
    )
    combine_proj = pl.pallas_call(
        _tc_combine_proj,
        out_shape=[jax.ShapeDtypeStruct((_N, _CH), _f32),
                   jax.ShapeDtypeStruct((_N, _CH), _f32)],
    )
    combine_final = pl.pallas_call(
        _tc_combine_final,
        out_shape=jax.ShapeDtypeStruct((_N, 128), _f32),
    )

    agg_deg = _sc_aggregate(True)
    agg_only = _sc_aggregate(False)

    xs1, hr1 = two_proj(x, W1l.T, W1r.T)
    agg1, deg = agg_deg(xs1, src2d, dst2d, zeros, ones)
    xs2, hr2 = combine_proj(agg1, deg, hr1, b1.reshape(1, _CH), W2l.T, W2r.T)
    (agg2,) = agg_only(xs2, src2d, dst2d, zeros)
    xs3, hr3 = combine_proj(agg2, deg, hr2, b2.reshape(1, _CH), W3l.T, W3r.T)
    (agg3,) = agg_only(xs3, src2d, dst2d, zeros)
    out = combine_final(agg3, deg, hr3, b3.reshape(1, _CH), Wo.T,
                        bo.reshape(1, 128))
    return out
